# Initial kernel scaffold; baseline (speedup 1.0000x reference)
import jax
import jax.numpy as jnp
from jax.experimental import pallas as pl


def kernel(x, edge_index, edge_attr, Wl1, bl1, Wr1, br1, We1, att1, bias1, Wl2, bl2, Wr2, br2, We2, att2, bias2, Wh1, bh1, Wh2, bh2):
    raise NotImplementedError("write your pallas kernel here")


# speedup vs baseline: 21.9020x; 21.9020x over previous
import functools

import jax
import jax.numpy as jnp
from jax import lax
from jax.experimental import pallas as pl
from jax.experimental.pallas import tpu as pltpu
from jax.experimental.pallas import tpu_sc as plsc

_N = 10000
_E = 320000
_EPS = 1e-16
_SLOPE = 0.2

_NWORK = 32
_PERW = _E // _NWORK
_K = 80
_NCHUNK = _PERW // _K
_NPAD = 10240
_RPT = _NPAD // 16
_RW = 128

_BN = 400
_BE = 4000


def _lin_kernel(x_ref, wl_ref, bl_ref, wr_ref, br_ref, xl_ref, xr_ref):
    xv = x_ref[...]
    xl_ref[...] = jnp.dot(xv, wl_ref[...], preferred_element_type=jnp.float32) + bl_ref[...]
    xr_ref[...] = jnp.dot(xv, wr_ref[...], preferred_element_type=jnp.float32) + br_ref[...]


def _lin_call(x, wl, bl, wr, br):
    din, dout = wl.shape
    return pl.pallas_call(
        _lin_kernel,
        grid=(_N // _BN,),
        in_specs=[
            pl.BlockSpec((_BN, din), lambda i: (i, 0)),
            pl.BlockSpec((din, dout), lambda i: (0, 0)),
            pl.BlockSpec((1, dout), lambda i: (0, 0)),
            pl.BlockSpec((din, dout), lambda i: (0, 0)),
            pl.BlockSpec((1, dout), lambda i: (0, 0)),
        ],
        out_specs=[
            pl.BlockSpec((_BN, dout), lambda i: (i, 0)),
            pl.BlockSpec((_BN, dout), lambda i: (i, 0)),
        ],
        out_shape=[
            jax.ShapeDtypeStruct((_N, dout), jnp.float32),
            jax.ShapeDtypeStruct((_N, dout), jnp.float32),
        ],
    )(x, wl, bl, wr, br)


def _gather_call(xl, xr, src, dst):
    d = 128
    mesh = plsc.VectorSubcoreMesh(core_axis_name="c", subcore_axis_name="s")

    @functools.partial(
        pl.kernel, mesh=mesh,
        out_type=(jax.ShapeDtypeStruct((_E, d), jnp.float32),
                  jax.ShapeDtypeStruct((_E, d), jnp.float32)),
        scratch_types=[
            pltpu.VMEM((_K,), jnp.int32),
            pltpu.VMEM((_K,), jnp.int32),
            pltpu.VMEM((_K, d), jnp.float32),
            pltpu.VMEM((_K, d), jnp.float32),
            pltpu.SemaphoreType.DMA,
            pltpu.SemaphoreType.DMA,
        ],
    )
    def gather_kernel(xl_hbm, xr_hbm, src_hbm, dst_hbm, gxl_hbm, gxr_hbm,
                      srcv, dstv, xlv, xrv, sem1, sem2):
        cid = lax.axis_index("c")
        sid = lax.axis_index("s")
        base0 = (cid * 16 + sid) * _PERW

        def chunk(ci, carry):
            b = base0 + ci * _K
            pltpu.sync_copy(src_hbm.at[pl.ds(b, _K)], srcv)
            pltpu.sync_copy(dst_hbm.at[pl.ds(b, _K)], dstv)
            c1 = pltpu.async_copy(xl_hbm.at[srcv], xlv, sem1)
            c2 = pltpu.async_copy(xr_hbm.at[dstv], xrv, sem2)
            c1.wait()
            c2.wait()
            pltpu.sync_copy(xlv, gxl_hbm.at[pl.ds(b, _K)])
            pltpu.sync_copy(xrv, gxr_hbm.at[pl.ds(b, _K)])
            return carry

        lax.fori_loop(0, _NCHUNK, chunk, 0)

    return gather_kernel(xl, xr, src, dst)


def _scatter_call(wm, dst):
    d = 128
    mesh = plsc.VectorSubcoreMesh(core_axis_name="c", subcore_axis_name="s")

    @functools.partial(
        pl.kernel, mesh=mesh,
        out_type=jax.ShapeDtypeStruct((2, _NPAD, d), jnp.float32),
        scratch_types=[
            pltpu.VMEM((_K,), jnp.int32),
            pltpu.VMEM((_K, d), jnp.float32),
            pltpu.VMEM((_RW, d), jnp.float32),
            pltpu.VMEM_SHARED((_NPAD, d), jnp.float32),
        ],
    )
    def scatter_kernel(wm_hbm, dst_hbm, zw_hbm, wp_hbm, dstv, wv, wbv, shw):
        cid = lax.axis_index("c")
        sid = lax.axis_index("s")
        r0 = sid * _RPT

        pltpu.sync_copy(zw_hbm, wbv)

        def zbody(t, carry):
            pltpu.sync_copy(wbv, shw.at[pl.ds(r0 + t * _RW, _RW)])
            return carry

        lax.fori_loop(0, _RPT // _RW, zbody, 0)
        plsc.subcore_barrier()

        base0 = (cid * 16 + sid) * _PERW

        def chunk(ci, carry):
            b = base0 + ci * _K
            pltpu.sync_copy(dst_hbm.at[pl.ds(b, _K)], dstv)
            pltpu.sync_copy(wm_hbm.at[pl.ds(b, _K)], wv)
            pltpu.sync_copy(wv, shw.at[dstv], add=True)
            return carry

        lax.fori_loop(0, _NCHUNK, chunk, 0)
        plsc.subcore_barrier()

        def wb_body(t, carry):
            r = r0 + t * _RW
            pltpu.sync_copy(shw.at[pl.ds(r, _RW)], wbv)
            pltpu.sync_copy(wbv, wp_hbm.at[cid, pl.ds(r, _RW)])
            return carry

        lax.fori_loop(0, _RPT // _RW, wb_body, 0)

    zw = jnp.zeros((_RW, d), jnp.float32)
    wp = scatter_kernel(wm, dst, zw)
    return wp[:, :_N]


def _edge1_kernel(gxl_ref, gxr_ref, ea_ref, we_ref, att_ref,
                  ma_ref, mb_ref, easum_ref):
    i = pl.program_id(0)
    ea = ea_ref[...]
    eaw = jnp.dot(ea, we_ref[...], preferred_element_type=jnp.float32)
    gxl = gxl_ref[...]
    u = gxl + gxr_ref[...] + eaw
    lr = jnp.maximum(u, _SLOPE * u)
    prod = lr * att_ref[...]
    e0 = jnp.exp(jnp.sum(prod[:, :64], axis=1, keepdims=True))
    e1 = jnp.exp(jnp.sum(prod[:, 64:], axis=1, keepdims=True))
    pad = jnp.zeros((e0.shape[0], 63), jnp.float32)
    ma_ref[...] = jnp.concatenate([e0 * gxl[:, :64], e0, pad], axis=1)
    mb_ref[...] = jnp.concatenate([e1 * gxl[:, 64:], e1, pad], axis=1)

    @pl.when(i == 0)
    def _():
        easum_ref[...] = jnp.zeros_like(easum_ref)

    easum_ref[...] += jnp.sum(ea, axis=0, keepdims=True)


def _edge1_call(gxl, gxr, ea, we, att):
    return pl.pallas_call(
        _edge1_kernel,
        grid=(_E // _BE,),
        in_specs=[
            pl.BlockSpec((_BE, 128), lambda i: (i, 0)),
            pl.BlockSpec((_BE, 128), lambda i: (i, 0)),
            pl.BlockSpec((_BE, 4), lambda i: (i, 0)),
            pl.BlockSpec((4, 128), lambda i: (0, 0)),
            pl.BlockSpec((1, 128), lambda i: (0, 0)),
        ],
        out_specs=[
            pl.BlockSpec((_BE, 128), lambda i: (i, 0)),
            pl.BlockSpec((_BE, 128), lambda i: (i, 0)),
            pl.BlockSpec((1, 4), lambda i: (0, 0)),
        ],
        out_shape=[
            jax.ShapeDtypeStruct((_E, 128), jnp.float32),
            jax.ShapeDtypeStruct((_E, 128), jnp.float32),
            jax.ShapeDtypeStruct((1, 4), jnp.float32),
        ],
    )(gxl, gxr, ea, we, att)


def _edge2_kernel(gsrc_ref, gdst_ref, ea_ref, we_ref, att_ref, m_ref):
    eaw = jnp.dot(ea_ref[...], we_ref[...], preferred_element_type=jnp.float32)
    xls = gsrc_ref[:, :64]
    u = xls + gdst_ref[:, 64:] + eaw
    lr = jnp.maximum(u, _SLOPE * u)
    e0 = jnp.exp(jnp.sum(lr * att_ref[...], axis=1, keepdims=True))
    pad = jnp.zeros((e0.shape[0], 63), jnp.float32)
    m_ref[...] = jnp.concatenate([e0 * xls, e0, pad], axis=1)


def _edge2_call(gsrc, gdst, ea, we, att):
    return pl.pallas_call(
        _edge2_kernel,
        grid=(_E // _BE,),
        in_specs=[
            pl.BlockSpec((_BE, 128), lambda i: (i, 0)),
            pl.BlockSpec((_BE, 128), lambda i: (i, 0)),
            pl.BlockSpec((_BE, 4), lambda i: (i, 0)),
            pl.BlockSpec((4, 64), lambda i: (0, 0)),
            pl.BlockSpec((1, 64), lambda i: (0, 0)),
        ],
        out_specs=[pl.BlockSpec((_BE, 128), lambda i: (i, 0))],
        out_shape=[jax.ShapeDtypeStruct((_E, 128), jnp.float32)],
    )(gsrc, gdst, ea, we, att)


def _merge1_kernel(pa_ref, pb_ref, xl_ref, xr_ref, easum_ref,
                   we_ref, att_ref, bias_ref, wl2_ref, bl2_ref, wr2_ref,
                   br2_ref, t2_ref):
    eawm = jnp.dot(easum_ref[...] / _E, we_ref[...],
                   preferred_element_type=jnp.float32)
    xl = xl_ref[...]
    u = xl + xr_ref[...] + eawm
    lr = jnp.maximum(u, _SLOPE * u)
    prod = lr * att_ref[...]
    e0 = jnp.exp(jnp.sum(prod[:, :64], axis=1, keepdims=True))
    e1 = jnp.exp(jnp.sum(prod[:, 64:], axis=1, keepdims=True))
    pa = pa_ref[0] + pa_ref[1]
    pb = pb_ref[0] + pb_ref[1]
    w0 = pa[:, :64] + e0 * xl[:, :64]
    w1 = pb[:, :64] + e1 * xl[:, 64:]
    s0 = pa[:, 64:65] + e0
    s1 = pb[:, 64:65] + e1
    h = jnp.concatenate([w0 / (s0 + _EPS), w1 / (s1 + _EPS)], axis=1)
    h = jnp.maximum(h + bias_ref[...], 0.0)
    xl2 = jnp.dot(h, wl2_ref[...], preferred_element_type=jnp.float32) + bl2_ref[...]
    xr2 = jnp.dot(h, wr2_ref[...], preferred_element_type=jnp.float32) + br2_ref[...]
    t2_ref[...] = jnp.concatenate([xl2, xr2], axis=1)


def _merge1_call(pa, pb, xl, xr, easum, we, att, bias, wl2, bl2,
                 wr2, br2):
    return pl.pallas_call(
        _merge1_kernel,
        grid=(_N // _BN,),
        in_specs=[
            pl.BlockSpec((2, _BN, 128), lambda i: (0, i, 0)),
            pl.BlockSpec((2, _BN, 128), lambda i: (0, i, 0)),
            pl.BlockSpec((_BN, 128), lambda i: (i, 0)),
            pl.BlockSpec((_BN, 128), lambda i: (i, 0)),
            pl.BlockSpec((1, 4), lambda i: (0, 0)),
            pl.BlockSpec((4, 128), lambda i: (0, 0)),
            pl.BlockSpec((1, 128), lambda i: (0, 0)),
            pl.BlockSpec((1, 128), lambda i: (0, 0)),
            pl.BlockSpec((128, 64), lambda i: (0, 0)),
            pl.BlockSpec((1, 64), lambda i: (0, 0)),
            pl.BlockSpec((128, 64), lambda i: (0, 0)),
            pl.BlockSpec((1, 64), lambda i: (0, 0)),
        ],
        out_specs=[pl.BlockSpec((_BN, 128), lambda i: (i, 0))],
        out_shape=[jax.ShapeDtypeStruct((_N, 128), jnp.float32)],
    )(pa, pb, xl, xr, easum, we, att, bias, wl2, bl2, wr2, br2)


def _merge2_kernel(p_ref, t2_ref, easum_ref, we_ref, att_ref,
                   bias_ref, wh1_ref, bh1_ref, wh2_ref, bh2_ref, out_ref):
    eawm = jnp.dot(easum_ref[...] / _E, we_ref[...],
                   preferred_element_type=jnp.float32)
    xl = t2_ref[:, :64]
    u = xl + t2_ref[:, 64:] + eawm
    lr = jnp.maximum(u, _SLOPE * u)
    e0 = jnp.exp(jnp.sum(lr * att_ref[...], axis=1, keepdims=True))
    p = p_ref[0] + p_ref[1]
    w = p[:, :64] + e0 * xl
    s0 = p[:, 64:65] + e0
    h = jnp.maximum(w / (s0 + _EPS) + bias_ref[...], 0.0)
    h = jnp.maximum(jnp.dot(h, wh1_ref[...], preferred_element_type=jnp.float32)
                    + bh1_ref[...], 0.0)
    out_ref[...] = jnp.sum(h * wh2_ref[...], axis=1, keepdims=True) + bh2_ref[...]


def _merge2_call(p, t2, easum, we, att, bias, wh1, bh1, wh2t, bh2):
    return pl.pallas_call(
        _merge2_kernel,
        grid=(_N // _BN,),
        in_specs=[
            pl.BlockSpec((2, _BN, 128), lambda i: (0, i, 0)),
            pl.BlockSpec((_BN, 128), lambda i: (i, 0)),
            pl.BlockSpec((1, 4), lambda i: (0, 0)),
            pl.BlockSpec((4, 64), lambda i: (0, 0)),
            pl.BlockSpec((1, 64), lambda i: (0, 0)),
            pl.BlockSpec((1, 64), lambda i: (0, 0)),
            pl.BlockSpec((64, 64), lambda i: (0, 0)),
            pl.BlockSpec((1, 64), lambda i: (0, 0)),
            pl.BlockSpec((1, 64), lambda i: (0, 0)),
            pl.BlockSpec((1, 1), lambda i: (0, 0)),
        ],
        out_specs=[pl.BlockSpec((_BN, 1), lambda i: (i, 0))],
        out_shape=[jax.ShapeDtypeStruct((_N, 1), jnp.float32)],
    )(p, t2, easum, we, att, bias, wh1, bh1, wh2t, bh2)


def kernel(x, edge_index, edge_attr, Wl1, bl1, Wr1, br1, We1, att1, bias1,
           Wl2, bl2, Wr2, br2, We2, att2, bias2, Wh1, bh1, Wh2, bh2):
    src = edge_index[0]
    dst = edge_index[1]

    xl1, xr1 = _lin_call(x, Wl1, bl1.reshape(1, -1), Wr1, br1.reshape(1, -1))
    gxl, gxr = _gather_call(xl1, xr1, src, dst)
    ma, mb, easum = _edge1_call(gxl, gxr, edge_attr, We1,
                                att1.reshape(1, -1))
    pa = _scatter_call(ma, dst)
    pb = _scatter_call(mb, dst)
    t2 = _merge1_call(pa, pb, xl1, xr1, easum, We1,
                      att1.reshape(1, -1), bias1.reshape(1, -1),
                      Wl2, bl2.reshape(1, -1), Wr2, br2.reshape(1, -1))[0]

    gsrc, gdst = _gather_call(t2, t2, src, dst)
    m2 = _edge2_call(gsrc, gdst, edge_attr, We2, att2.reshape(1, -1))[0]
    p2 = _scatter_call(m2, dst)
    out = _merge2_call(p2, t2, easum, We2,
                       att2.reshape(1, -1), bias2.reshape(1, -1),
                       Wh1, bh1.reshape(1, -1), Wh2.reshape(1, -1),
                       bh2.reshape(1, 1))
    return out[0][:, 0]


# speedup vs baseline: 30.6992x; 1.4017x over previous
import functools

import jax
import jax.numpy as jnp
from jax import lax
from jax.experimental import pallas as pl
from jax.experimental.pallas import tpu as pltpu
from jax.experimental.pallas import tpu_sc as plsc

_N = 10000
_E = 320000
_EPS = 1e-16
_SLOPE = 0.2

_NWORK = 32
_PERW = _E // _NWORK
_K = 80
_NCHUNK = _PERW // _K
_NPAD = 10240
_RPT = _NPAD // 16
_RW = 128

_BN = 400
_BE = 4000


def _lin_kernel(x_ref, wl_ref, bl_ref, wr_ref, br_ref, xl_ref, xr_ref):
    xv = x_ref[...]
    xl_ref[...] = jnp.dot(xv, wl_ref[...], preferred_element_type=jnp.float32) + bl_ref[...]
    xr_ref[...] = jnp.dot(xv, wr_ref[...], preferred_element_type=jnp.float32) + br_ref[...]


def _lin_call(x, wl, bl, wr, br):
    din, dout = wl.shape
    return pl.pallas_call(
        _lin_kernel,
        grid=(_N // _BN,),
        in_specs=[
            pl.BlockSpec((_BN, din), lambda i: (i, 0)),
            pl.BlockSpec((din, dout), lambda i: (0, 0)),
            pl.BlockSpec((1, dout), lambda i: (0, 0)),
            pl.BlockSpec((din, dout), lambda i: (0, 0)),
            pl.BlockSpec((1, dout), lambda i: (0, 0)),
        ],
        out_specs=[
            pl.BlockSpec((_BN, dout), lambda i: (i, 0)),
            pl.BlockSpec((_BN, dout), lambda i: (i, 0)),
        ],
        out_shape=[
            jax.ShapeDtypeStruct((_N, dout), jnp.float32),
            jax.ShapeDtypeStruct((_N, dout), jnp.float32),
        ],
    )(x, wl, bl, wr, br)


_GG = 4
_NGG = _NCHUNK // _GG


def _gather_call(xl, xr, src, dst):
    d = 128
    mesh = plsc.VectorSubcoreMesh(core_axis_name="c", subcore_axis_name="s")

    scratch = [
        pltpu.VMEM((_PERW,), jnp.int32),
        pltpu.VMEM((_PERW,), jnp.int32),
    ]
    scratch += [pltpu.VMEM((_K, d), jnp.float32) for _ in range(2 * _GG)]
    scratch += [pltpu.SemaphoreType.DMA, pltpu.SemaphoreType.DMA]

    @functools.partial(
        pl.kernel, mesh=mesh,
        out_type=(jax.ShapeDtypeStruct((_E, d), jnp.float32),
                  jax.ShapeDtypeStruct((_E, d), jnp.float32)),
        scratch_types=scratch,
    )
    def gather_kernel(xl_hbm, xr_hbm, src_hbm, dst_hbm, gxl_hbm, gxr_hbm,
                      *bufs):
        srcall, dstall = bufs[0], bufs[1]
        xlv = bufs[2:2 + _GG]
        xrv = bufs[2 + _GG:2 + 2 * _GG]
        semg, semw = bufs[-2], bufs[-1]
        cid = lax.axis_index("c")
        sid = lax.axis_index("s")
        base0 = (cid * 16 + sid) * _PERW

        pltpu.sync_copy(src_hbm.at[pl.ds(base0, _PERW)], srcall)
        pltpu.sync_copy(dst_hbm.at[pl.ds(base0, _PERW)], dstall)

        def do_group(c0, n):
            gd = []
            for s in range(n):
                o = (c0 + s) * _K
                gd.append(pltpu.async_copy(
                    xl_hbm.at[srcall.at[pl.ds(o, _K)]], xlv[s], semg))
                gd.append(pltpu.async_copy(
                    xr_hbm.at[dstall.at[pl.ds(o, _K)]], xrv[s], semg))
            for c in gd:
                c.wait()
            wd = []
            for s in range(n):
                b = base0 + (c0 + s) * _K
                wd.append(pltpu.async_copy(xlv[s], gxl_hbm.at[pl.ds(b, _K)], semw))
                wd.append(pltpu.async_copy(xrv[s], gxr_hbm.at[pl.ds(b, _K)], semw))
            for c in wd:
                c.wait()

        def group(g, carry):
            do_group(g * _GG, _GG)
            return carry

        lax.fori_loop(0, _NGG, group, 0)
        if _NCHUNK % _GG:
            do_group(_NGG * _GG, _NCHUNK % _GG)

    return gather_kernel(xl, xr, src, dst)


def _scatter_call(wm, dst):
    d = 128
    mesh = plsc.VectorSubcoreMesh(core_axis_name="c", subcore_axis_name="s")

    @functools.partial(
        pl.kernel, mesh=mesh,
        out_type=jax.ShapeDtypeStruct((2, _NPAD, d), jnp.float32),
        scratch_types=[
            pltpu.VMEM((_K,), jnp.int32),
            pltpu.VMEM((_K,), jnp.int32),
            pltpu.VMEM((_K, d), jnp.float32),
            pltpu.VMEM((_K, d), jnp.float32),
            pltpu.VMEM((_RW, d), jnp.float32),
            pltpu.VMEM_SHARED((_NPAD, d), jnp.float32),
            pltpu.SemaphoreType.DMA,
        ],
    )
    def scatter_kernel(wm_hbm, dst_hbm, zw_hbm, wp_hbm,
                       dstv0, dstv1, wv0, wv1, wbv, shw, semr):
        cid = lax.axis_index("c")
        sid = lax.axis_index("s")
        r0 = sid * _RPT

        pltpu.sync_copy(zw_hbm, wbv)

        def zbody(t, carry):
            pltpu.sync_copy(wbv, shw.at[pl.ds(r0 + t * _RW, _RW)])
            return carry

        lax.fori_loop(0, _RPT // _RW, zbody, 0)
        plsc.subcore_barrier()

        base0 = (cid * 16 + sid) * _PERW

        def do_pair(c0, pair):
            descs = []
            for s, (dv, wv) in enumerate(pair):
                b = base0 + (c0 + s) * _K
                descs.append(pltpu.async_copy(dst_hbm.at[pl.ds(b, _K)], dv, semr))
                descs.append(pltpu.async_copy(wm_hbm.at[pl.ds(b, _K)], wv, semr))
            for i, (dv, wv) in enumerate(pair):
                descs[2 * i].wait()
                descs[2 * i + 1].wait()
                pltpu.sync_copy(wv, shw.at[dv], add=True)

        def chunk(g, carry):
            do_pair(g * 2, [(dstv0, wv0), (dstv1, wv1)])
            return carry

        lax.fori_loop(0, _NCHUNK // 2, chunk, 0)
        if _NCHUNK % 2:
            do_pair(_NCHUNK - 1, [(dstv0, wv0)])
        plsc.subcore_barrier()

        def wb_body(t, carry):
            r = r0 + t * _RW
            pltpu.sync_copy(shw.at[pl.ds(r, _RW)], wbv)
            pltpu.sync_copy(wbv, wp_hbm.at[cid, pl.ds(r, _RW)])
            return carry

        lax.fori_loop(0, _RPT // _RW, wb_body, 0)

    zw = jnp.zeros((_RW, d), jnp.float32)
    wp = scatter_kernel(wm, dst, zw)
    return wp[:, :_N]


def _edge1_kernel(gxl_ref, gxr_ref, ea_ref, we_ref, att_ref,
                  ma_ref, mb_ref, easum_ref):
    i = pl.program_id(0)
    ea = ea_ref[...]
    eaw = jnp.dot(ea, we_ref[...], preferred_element_type=jnp.float32)
    gxl = gxl_ref[...]
    u = gxl + gxr_ref[...] + eaw
    lr = jnp.maximum(u, _SLOPE * u)
    prod = lr * att_ref[...]
    e0 = jnp.exp(jnp.sum(prod[:, :64], axis=1, keepdims=True))
    e1 = jnp.exp(jnp.sum(prod[:, 64:], axis=1, keepdims=True))
    pad = jnp.zeros((e0.shape[0], 63), jnp.float32)
    ma_ref[...] = jnp.concatenate([e0 * gxl[:, :64], e0, pad], axis=1)
    mb_ref[...] = jnp.concatenate([e1 * gxl[:, 64:], e1, pad], axis=1)

    @pl.when(i == 0)
    def _():
        easum_ref[...] = jnp.zeros_like(easum_ref)

    easum_ref[...] += jnp.sum(ea, axis=0, keepdims=True)


def _edge1_call(gxl, gxr, ea, we, att):
    return pl.pallas_call(
        _edge1_kernel,
        grid=(_E // _BE,),
        in_specs=[
            pl.BlockSpec((_BE, 128), lambda i: (i, 0)),
            pl.BlockSpec((_BE, 128), lambda i: (i, 0)),
            pl.BlockSpec((_BE, 4), lambda i: (i, 0)),
            pl.BlockSpec((4, 128), lambda i: (0, 0)),
            pl.BlockSpec((1, 128), lambda i: (0, 0)),
        ],
        out_specs=[
            pl.BlockSpec((_BE, 128), lambda i: (i, 0)),
            pl.BlockSpec((_BE, 128), lambda i: (i, 0)),
            pl.BlockSpec((1, 4), lambda i: (0, 0)),
        ],
        out_shape=[
            jax.ShapeDtypeStruct((_E, 128), jnp.float32),
            jax.ShapeDtypeStruct((_E, 128), jnp.float32),
            jax.ShapeDtypeStruct((1, 4), jnp.float32),
        ],
    )(gxl, gxr, ea, we, att)


def _edge2_kernel(gsrc_ref, gdst_ref, ea_ref, we_ref, att_ref, m_ref):
    eaw = jnp.dot(ea_ref[...], we_ref[...], preferred_element_type=jnp.float32)
    xls = gsrc_ref[:, :64]
    u = xls + gdst_ref[:, 64:] + eaw
    lr = jnp.maximum(u, _SLOPE * u)
    e0 = jnp.exp(jnp.sum(lr * att_ref[...], axis=1, keepdims=True))
    pad = jnp.zeros((e0.shape[0], 63), jnp.float32)
    m_ref[...] = jnp.concatenate([e0 * xls, e0, pad], axis=1)


def _edge2_call(gsrc, gdst, ea, we, att):
    return pl.pallas_call(
        _edge2_kernel,
        grid=(_E // _BE,),
        in_specs=[
            pl.BlockSpec((_BE, 128), lambda i: (i, 0)),
            pl.BlockSpec((_BE, 128), lambda i: (i, 0)),
            pl.BlockSpec((_BE, 4), lambda i: (i, 0)),
            pl.BlockSpec((4, 64), lambda i: (0, 0)),
            pl.BlockSpec((1, 64), lambda i: (0, 0)),
        ],
        out_specs=[pl.BlockSpec((_BE, 128), lambda i: (i, 0))],
        out_shape=[jax.ShapeDtypeStruct((_E, 128), jnp.float32)],
    )(gsrc, gdst, ea, we, att)


def _merge1_kernel(pa_ref, pb_ref, xl_ref, xr_ref, easum_ref,
                   we_ref, att_ref, bias_ref, wl2_ref, bl2_ref, wr2_ref,
                   br2_ref, t2_ref):
    eawm = jnp.dot(easum_ref[...] / _E, we_ref[...],
                   preferred_element_type=jnp.float32)
    xl = xl_ref[...]
    u = xl + xr_ref[...] + eawm
    lr = jnp.maximum(u, _SLOPE * u)
    prod = lr * att_ref[...]
    e0 = jnp.exp(jnp.sum(prod[:, :64], axis=1, keepdims=True))
    e1 = jnp.exp(jnp.sum(prod[:, 64:], axis=1, keepdims=True))
    pa = pa_ref[0] + pa_ref[1]
    pb = pb_ref[0] + pb_ref[1]
    w0 = pa[:, :64] + e0 * xl[:, :64]
    w1 = pb[:, :64] + e1 * xl[:, 64:]
    s0 = pa[:, 64:65] + e0
    s1 = pb[:, 64:65] + e1
    h = jnp.concatenate([w0 / (s0 + _EPS), w1 / (s1 + _EPS)], axis=1)
    h = jnp.maximum(h + bias_ref[...], 0.0)
    xl2 = jnp.dot(h, wl2_ref[...], preferred_element_type=jnp.float32) + bl2_ref[...]
    xr2 = jnp.dot(h, wr2_ref[...], preferred_element_type=jnp.float32) + br2_ref[...]
    t2_ref[...] = jnp.concatenate([xl2, xr2], axis=1)


def _merge1_call(pa, pb, xl, xr, easum, we, att, bias, wl2, bl2,
                 wr2, br2):
    return pl.pallas_call(
        _merge1_kernel,
        grid=(_N // _BN,),
        in_specs=[
            pl.BlockSpec((2, _BN, 128), lambda i: (0, i, 0)),
            pl.BlockSpec((2, _BN, 128), lambda i: (0, i, 0)),
            pl.BlockSpec((_BN, 128), lambda i: (i, 0)),
            pl.BlockSpec((_BN, 128), lambda i: (i, 0)),
            pl.BlockSpec((1, 4), lambda i: (0, 0)),
            pl.BlockSpec((4, 128), lambda i: (0, 0)),
            pl.BlockSpec((1, 128), lambda i: (0, 0)),
            pl.BlockSpec((1, 128), lambda i: (0, 0)),
            pl.BlockSpec((128, 64), lambda i: (0, 0)),
            pl.BlockSpec((1, 64), lambda i: (0, 0)),
            pl.BlockSpec((128, 64), lambda i: (0, 0)),
            pl.BlockSpec((1, 64), lambda i: (0, 0)),
        ],
        out_specs=[pl.BlockSpec((_BN, 128), lambda i: (i, 0))],
        out_shape=[jax.ShapeDtypeStruct((_N, 128), jnp.float32)],
    )(pa, pb, xl, xr, easum, we, att, bias, wl2, bl2, wr2, br2)


def _merge2_kernel(p_ref, t2_ref, easum_ref, we_ref, att_ref,
                   bias_ref, wh1_ref, bh1_ref, wh2_ref, bh2_ref, out_ref):
    eawm = jnp.dot(easum_ref[...] / _E, we_ref[...],
                   preferred_element_type=jnp.float32)
    xl = t2_ref[:, :64]
    u = xl + t2_ref[:, 64:] + eawm
    lr = jnp.maximum(u, _SLOPE * u)
    e0 = jnp.exp(jnp.sum(lr * att_ref[...], axis=1, keepdims=True))
    p = p_ref[0] + p_ref[1]
    w = p[:, :64] + e0 * xl
    s0 = p[:, 64:65] + e0
    h = jnp.maximum(w / (s0 + _EPS) + bias_ref[...], 0.0)
    h = jnp.maximum(jnp.dot(h, wh1_ref[...], preferred_element_type=jnp.float32)
                    + bh1_ref[...], 0.0)
    out_ref[...] = jnp.sum(h * wh2_ref[...], axis=1, keepdims=True) + bh2_ref[...]


def _merge2_call(p, t2, easum, we, att, bias, wh1, bh1, wh2t, bh2):
    return pl.pallas_call(
        _merge2_kernel,
        grid=(_N // _BN,),
        in_specs=[
            pl.BlockSpec((2, _BN, 128), lambda i: (0, i, 0)),
            pl.BlockSpec((_BN, 128), lambda i: (i, 0)),
            pl.BlockSpec((1, 4), lambda i: (0, 0)),
            pl.BlockSpec((4, 64), lambda i: (0, 0)),
            pl.BlockSpec((1, 64), lambda i: (0, 0)),
            pl.BlockSpec((1, 64), lambda i: (0, 0)),
            pl.BlockSpec((64, 64), lambda i: (0, 0)),
            pl.BlockSpec((1, 64), lambda i: (0, 0)),
            pl.BlockSpec((1, 64), lambda i: (0, 0)),
            pl.BlockSpec((1, 1), lambda i: (0, 0)),
        ],
        out_specs=[pl.BlockSpec((_BN, 1), lambda i: (i, 0))],
        out_shape=[jax.ShapeDtypeStruct((_N, 1), jnp.float32)],
    )(p, t2, easum, we, att, bias, wh1, bh1, wh2t, bh2)


def kernel(x, edge_index, edge_attr, Wl1, bl1, Wr1, br1, We1, att1, bias1,
           Wl2, bl2, Wr2, br2, We2, att2, bias2, Wh1, bh1, Wh2, bh2):
    src = edge_index[0]
    dst = edge_index[1]

    xl1, xr1 = _lin_call(x, Wl1, bl1.reshape(1, -1), Wr1, br1.reshape(1, -1))
    gxl, gxr = _gather_call(xl1, xr1, src, dst)
    ma, mb, easum = _edge1_call(gxl, gxr, edge_attr, We1,
                                att1.reshape(1, -1))
    pa = _scatter_call(ma, dst)
    pb = _scatter_call(mb, dst)
    t2 = _merge1_call(pa, pb, xl1, xr1, easum, We1,
                      att1.reshape(1, -1), bias1.reshape(1, -1),
                      Wl2, bl2.reshape(1, -1), Wr2, br2.reshape(1, -1))[0]

    gsrc, gdst = _gather_call(t2, t2, src, dst)
    m2 = _edge2_call(gsrc, gdst, edge_attr, We2, att2.reshape(1, -1))[0]
    p2 = _scatter_call(m2, dst)
    out = _merge2_call(p2, t2, easum, We2,
                       att2.reshape(1, -1), bias2.reshape(1, -1),
                       Wh1, bh1.reshape(1, -1), Wh2.reshape(1, -1),
                       bh2.reshape(1, 1))
    return out[0][:, 0]


# speedup vs baseline: 30.9977x; 1.0097x over previous
import functools

import jax
import jax.numpy as jnp
from jax import lax
from jax.experimental import pallas as pl
from jax.experimental.pallas import tpu as pltpu
from jax.experimental.pallas import tpu_sc as plsc

_N = 10000
_E = 320000
_EPS = 1e-16
_SLOPE = 0.2

_NWORK = 32
_PERW = _E // _NWORK
_K = 80
_NCHUNK = _PERW // _K
_NPAD = 10240
_RPT = _NPAD // 16
_RW = 128

_BN = 400
_BE = 4000


def _lin_kernel(x_ref, wl_ref, bl_ref, wr_ref, br_ref, xl_ref, xr_ref):
    xv = x_ref[...]
    xl_ref[...] = jnp.dot(xv, wl_ref[...], preferred_element_type=jnp.float32) + bl_ref[...]
    xr_ref[...] = jnp.dot(xv, wr_ref[...], preferred_element_type=jnp.float32) + br_ref[...]


def _lin_call(x, wl, bl, wr, br):
    din, dout = wl.shape
    return pl.pallas_call(
        _lin_kernel,
        grid=(_N // _BN,),
        in_specs=[
            pl.BlockSpec((_BN, din), lambda i: (i, 0)),
            pl.BlockSpec((din, dout), lambda i: (0, 0)),
            pl.BlockSpec((1, dout), lambda i: (0, 0)),
            pl.BlockSpec((din, dout), lambda i: (0, 0)),
            pl.BlockSpec((1, dout), lambda i: (0, 0)),
        ],
        out_specs=[
            pl.BlockSpec((_BN, dout), lambda i: (i, 0)),
            pl.BlockSpec((_BN, dout), lambda i: (i, 0)),
        ],
        out_shape=[
            jax.ShapeDtypeStruct((_N, dout), jnp.float32),
            jax.ShapeDtypeStruct((_N, dout), jnp.float32),
        ],
    )(x, wl, bl, wr, br)


_GG = 5
_NGG = _NCHUNK // _GG


def _gather_call(xl, xr, src, dst):
    d = 128
    mesh = plsc.VectorSubcoreMesh(core_axis_name="c", subcore_axis_name="s")

    scratch = [
        pltpu.VMEM((_PERW,), jnp.int32),
        pltpu.VMEM((_PERW,), jnp.int32),
    ]
    scratch += [pltpu.VMEM((_K, d), jnp.float32) for _ in range(2 * _GG)]
    scratch += [pltpu.SemaphoreType.DMA, pltpu.SemaphoreType.DMA]

    @functools.partial(
        pl.kernel, mesh=mesh,
        out_type=(jax.ShapeDtypeStruct((_E, d), jnp.float32),
                  jax.ShapeDtypeStruct((_E, d), jnp.float32)),
        scratch_types=scratch,
    )
    def gather_kernel(xl_hbm, xr_hbm, src_hbm, dst_hbm, gxl_hbm, gxr_hbm,
                      *bufs):
        srcall, dstall = bufs[0], bufs[1]
        xlv = bufs[2:2 + _GG]
        xrv = bufs[2 + _GG:2 + 2 * _GG]
        semg, semw = bufs[-2], bufs[-1]
        cid = lax.axis_index("c")
        sid = lax.axis_index("s")
        base0 = (cid * 16 + sid) * _PERW

        pltpu.sync_copy(src_hbm.at[pl.ds(base0, _PERW)], srcall)
        pltpu.sync_copy(dst_hbm.at[pl.ds(base0, _PERW)], dstall)

        def do_group(c0, n):
            gd = []
            for s in range(n):
                o = (c0 + s) * _K
                gd.append(pltpu.async_copy(
                    xl_hbm.at[srcall.at[pl.ds(o, _K)]], xlv[s], semg))
                gd.append(pltpu.async_copy(
                    xr_hbm.at[dstall.at[pl.ds(o, _K)]], xrv[s], semg))
            for c in gd:
                c.wait()
            wd = []
            for s in range(n):
                b = base0 + (c0 + s) * _K
                wd.append(pltpu.async_copy(xlv[s], gxl_hbm.at[pl.ds(b, _K)], semw))
                wd.append(pltpu.async_copy(xrv[s], gxr_hbm.at[pl.ds(b, _K)], semw))
            for c in wd:
                c.wait()

        def group(g, carry):
            do_group(g * _GG, _GG)
            return carry

        lax.fori_loop(0, _NGG, group, 0)
        if _NCHUNK % _GG:
            do_group(_NGG * _GG, _NCHUNK % _GG)

    return gather_kernel(xl, xr, src, dst)


def _scatter_call(wm, dst):
    d = 128
    mesh = plsc.VectorSubcoreMesh(core_axis_name="c", subcore_axis_name="s")

    @functools.partial(
        pl.kernel, mesh=mesh,
        out_type=jax.ShapeDtypeStruct((2, _NPAD, d), jnp.float32),
        scratch_types=[
            pltpu.VMEM((_K,), jnp.int32),
            pltpu.VMEM((_K,), jnp.int32),
            pltpu.VMEM((_K, d), jnp.float32),
            pltpu.VMEM((_K, d), jnp.float32),
            pltpu.VMEM((_RW, d), jnp.float32),
            pltpu.VMEM_SHARED((_NPAD, d), jnp.float32),
            pltpu.SemaphoreType.DMA,
            pltpu.SemaphoreType.DMA,
        ],
    )
    def scatter_kernel(wm_hbm, dst_hbm, zw_hbm, wp_hbm,
                       dstv0, dstv1, wv0, wv1, wbv, shw, semr, sema):
        cid = lax.axis_index("c")
        sid = lax.axis_index("s")
        r0 = sid * _RPT

        pltpu.sync_copy(zw_hbm, wbv)

        def zbody(t, carry):
            pltpu.sync_copy(wbv, shw.at[pl.ds(r0 + t * _RW, _RW)])
            return carry

        lax.fori_loop(0, _RPT // _RW, zbody, 0)
        plsc.subcore_barrier()

        base0 = (cid * 16 + sid) * _PERW

        def do_pair(c0, pair):
            descs = []
            for s, (dv, wv) in enumerate(pair):
                b = base0 + (c0 + s) * _K
                descs.append(pltpu.async_copy(dst_hbm.at[pl.ds(b, _K)], dv, semr))
                descs.append(pltpu.async_copy(wm_hbm.at[pl.ds(b, _K)], wv, semr))
            adds = []
            for i, (dv, wv) in enumerate(pair):
                descs[2 * i].wait()
                descs[2 * i + 1].wait()
                adds.append(pltpu.async_copy(wv, shw.at[dv], sema, add=True))
            for c in adds:
                c.wait()

        def chunk(g, carry):
            do_pair(g * 2, [(dstv0, wv0), (dstv1, wv1)])
            return carry

        lax.fori_loop(0, _NCHUNK // 2, chunk, 0)
        if _NCHUNK % 2:
            do_pair(_NCHUNK - 1, [(dstv0, wv0)])
        plsc.subcore_barrier()

        def wb_body(t, carry):
            r = r0 + t * _RW
            pltpu.sync_copy(shw.at[pl.ds(r, _RW)], wbv)
            pltpu.sync_copy(wbv, wp_hbm.at[cid, pl.ds(r, _RW)])
            return carry

        lax.fori_loop(0, _RPT // _RW, wb_body, 0)

    zw = jnp.zeros((_RW, d), jnp.float32)
    wp = scatter_kernel(wm, dst, zw)
    return wp[:, :_N]


def _edge1_kernel(gxl_ref, gxr_ref, ea_ref, we_ref, att_ref,
                  ma_ref, mb_ref, easum_ref):
    i = pl.program_id(0)
    ea = ea_ref[...]
    eaw = jnp.dot(ea, we_ref[...], preferred_element_type=jnp.float32)
    gxl = gxl_ref[...]
    u = gxl + gxr_ref[...] + eaw
    lr = jnp.maximum(u, _SLOPE * u)
    prod = lr * att_ref[...]
    e0 = jnp.exp(jnp.sum(prod[:, :64], axis=1, keepdims=True))
    e1 = jnp.exp(jnp.sum(prod[:, 64:], axis=1, keepdims=True))
    pad = jnp.zeros((e0.shape[0], 63), jnp.float32)
    ma_ref[...] = jnp.concatenate([e0 * gxl[:, :64], e0, pad], axis=1)
    mb_ref[...] = jnp.concatenate([e1 * gxl[:, 64:], e1, pad], axis=1)

    @pl.when(i == 0)
    def _():
        easum_ref[...] = jnp.zeros_like(easum_ref)

    easum_ref[...] += jnp.sum(ea, axis=0, keepdims=True)


def _edge1_call(gxl, gxr, ea, we, att):
    return pl.pallas_call(
        _edge1_kernel,
        grid=(_E // _BE,),
        in_specs=[
            pl.BlockSpec((_BE, 128), lambda i: (i, 0)),
            pl.BlockSpec((_BE, 128), lambda i: (i, 0)),
            pl.BlockSpec((_BE, 4), lambda i: (i, 0)),
            pl.BlockSpec((4, 128), lambda i: (0, 0)),
            pl.BlockSpec((1, 128), lambda i: (0, 0)),
        ],
        out_specs=[
            pl.BlockSpec((_BE, 128), lambda i: (i, 0)),
            pl.BlockSpec((_BE, 128), lambda i: (i, 0)),
            pl.BlockSpec((1, 4), lambda i: (0, 0)),
        ],
        out_shape=[
            jax.ShapeDtypeStruct((_E, 128), jnp.float32),
            jax.ShapeDtypeStruct((_E, 128), jnp.float32),
            jax.ShapeDtypeStruct((1, 4), jnp.float32),
        ],
    )(gxl, gxr, ea, we, att)


def _edge2_kernel(gsrc_ref, gdst_ref, ea_ref, we_ref, att_ref, m_ref):
    eaw = jnp.dot(ea_ref[...], we_ref[...], preferred_element_type=jnp.float32)
    xls = gsrc_ref[:, :64]
    u = xls + gdst_ref[:, 64:] + eaw
    lr = jnp.maximum(u, _SLOPE * u)
    e0 = jnp.exp(jnp.sum(lr * att_ref[...], axis=1, keepdims=True))
    pad = jnp.zeros((e0.shape[0], 63), jnp.float32)
    m_ref[...] = jnp.concatenate([e0 * xls, e0, pad], axis=1)


def _edge2_call(gsrc, gdst, ea, we, att):
    return pl.pallas_call(
        _edge2_kernel,
        grid=(_E // _BE,),
        in_specs=[
            pl.BlockSpec((_BE, 128), lambda i: (i, 0)),
            pl.BlockSpec((_BE, 128), lambda i: (i, 0)),
            pl.BlockSpec((_BE, 4), lambda i: (i, 0)),
            pl.BlockSpec((4, 64), lambda i: (0, 0)),
            pl.BlockSpec((1, 64), lambda i: (0, 0)),
        ],
        out_specs=[pl.BlockSpec((_BE, 128), lambda i: (i, 0))],
        out_shape=[jax.ShapeDtypeStruct((_E, 128), jnp.float32)],
    )(gsrc, gdst, ea, we, att)


def _merge1_kernel(pa_ref, pb_ref, xl_ref, xr_ref, easum_ref,
                   we_ref, att_ref, bias_ref, wl2_ref, bl2_ref, wr2_ref,
                   br2_ref, t2_ref):
    eawm = jnp.dot(easum_ref[...] / _E, we_ref[...],
                   preferred_element_type=jnp.float32)
    xl = xl_ref[...]
    u = xl + xr_ref[...] + eawm
    lr = jnp.maximum(u, _SLOPE * u)
    prod = lr * att_ref[...]
    e0 = jnp.exp(jnp.sum(prod[:, :64], axis=1, keepdims=True))
    e1 = jnp.exp(jnp.sum(prod[:, 64:], axis=1, keepdims=True))
    pa = pa_ref[0] + pa_ref[1]
    pb = pb_ref[0] + pb_ref[1]
    w0 = pa[:, :64] + e0 * xl[:, :64]
    w1 = pb[:, :64] + e1 * xl[:, 64:]
    s0 = pa[:, 64:65] + e0
    s1 = pb[:, 64:65] + e1
    h = jnp.concatenate([w0 / (s0 + _EPS), w1 / (s1 + _EPS)], axis=1)
    h = jnp.maximum(h + bias_ref[...], 0.0)
    xl2 = jnp.dot(h, wl2_ref[...], preferred_element_type=jnp.float32) + bl2_ref[...]
    xr2 = jnp.dot(h, wr2_ref[...], preferred_element_type=jnp.float32) + br2_ref[...]
    t2_ref[...] = jnp.concatenate([xl2, xr2], axis=1)


def _merge1_call(pa, pb, xl, xr, easum, we, att, bias, wl2, bl2,
                 wr2, br2):
    return pl.pallas_call(
        _merge1_kernel,
        grid=(_N // _BN,),
        in_specs=[
            pl.BlockSpec((2, _BN, 128), lambda i: (0, i, 0)),
            pl.BlockSpec((2, _BN, 128), lambda i: (0, i, 0)),
            pl.BlockSpec((_BN, 128), lambda i: (i, 0)),
            pl.BlockSpec((_BN, 128), lambda i: (i, 0)),
            pl.BlockSpec((1, 4), lambda i: (0, 0)),
            pl.BlockSpec((4, 128), lambda i: (0, 0)),
            pl.BlockSpec((1, 128), lambda i: (0, 0)),
            pl.BlockSpec((1, 128), lambda i: (0, 0)),
            pl.BlockSpec((128, 64), lambda i: (0, 0)),
            pl.BlockSpec((1, 64), lambda i: (0, 0)),
            pl.BlockSpec((128, 64), lambda i: (0, 0)),
            pl.BlockSpec((1, 64), lambda i: (0, 0)),
        ],
        out_specs=[pl.BlockSpec((_BN, 128), lambda i: (i, 0))],
        out_shape=[jax.ShapeDtypeStruct((_N, 128), jnp.float32)],
    )(pa, pb, xl, xr, easum, we, att, bias, wl2, bl2, wr2, br2)


def _merge2_kernel(p_ref, t2_ref, easum_ref, we_ref, att_ref,
                   bias_ref, wh1_ref, bh1_ref, wh2_ref, bh2_ref, out_ref):
    eawm = jnp.dot(easum_ref[...] / _E, we_ref[...],
                   preferred_element_type=jnp.float32)
    xl = t2_ref[:, :64]
    u = xl + t2_ref[:, 64:] + eawm
    lr = jnp.maximum(u, _SLOPE * u)
    e0 = jnp.exp(jnp.sum(lr * att_ref[...], axis=1, keepdims=True))
    p = p_ref[0] + p_ref[1]
    w = p[:, :64] + e0 * xl
    s0 = p[:, 64:65] + e0
    h = jnp.maximum(w / (s0 + _EPS) + bias_ref[...], 0.0)
    h = jnp.maximum(jnp.dot(h, wh1_ref[...], preferred_element_type=jnp.float32)
                    + bh1_ref[...], 0.0)
    out_ref[...] = jnp.sum(h * wh2_ref[...], axis=1, keepdims=True) + bh2_ref[...]


def _merge2_call(p, t2, easum, we, att, bias, wh1, bh1, wh2t, bh2):
    return pl.pallas_call(
        _merge2_kernel,
        grid=(_N // _BN,),
        in_specs=[
            pl.BlockSpec((2, _BN, 128), lambda i: (0, i, 0)),
            pl.BlockSpec((_BN, 128), lambda i: (i, 0)),
            pl.BlockSpec((1, 4), lambda i: (0, 0)),
            pl.BlockSpec((4, 64), lambda i: (0, 0)),
            pl.BlockSpec((1, 64), lambda i: (0, 0)),
            pl.BlockSpec((1, 64), lambda i: (0, 0)),
            pl.BlockSpec((64, 64), lambda i: (0, 0)),
            pl.BlockSpec((1, 64), lambda i: (0, 0)),
            pl.BlockSpec((1, 64), lambda i: (0, 0)),
            pl.BlockSpec((1, 1), lambda i: (0, 0)),
        ],
        out_specs=[pl.BlockSpec((_BN, 1), lambda i: (i, 0))],
        out_shape=[jax.ShapeDtypeStruct((_N, 1), jnp.float32)],
    )(p, t2, easum, we, att, bias, wh1, bh1, wh2t, bh2)


def kernel(x, edge_index, edge_attr, Wl1, bl1, Wr1, br1, We1, att1, bias1,
           Wl2, bl2, Wr2, br2, We2, att2, bias2, Wh1, bh1, Wh2, bh2):
    src = edge_index[0]
    dst = edge_index[1]

    xl1, xr1 = _lin_call(x, Wl1, bl1.reshape(1, -1), Wr1, br1.reshape(1, -1))
    gxl, gxr = _gather_call(xl1, xr1, src, dst)
    ma, mb, easum = _edge1_call(gxl, gxr, edge_attr, We1,
                                att1.reshape(1, -1))
    pa = _scatter_call(ma, dst)
    pb = _scatter_call(mb, dst)
    t2 = _merge1_call(pa, pb, xl1, xr1, easum, We1,
                      att1.reshape(1, -1), bias1.reshape(1, -1),
                      Wl2, bl2.reshape(1, -1), Wr2, br2.reshape(1, -1))[0]

    gsrc, gdst = _gather_call(t2, t2, src, dst)
    m2 = _edge2_call(gsrc, gdst, edge_attr, We2, att2.reshape(1, -1))[0]
    p2 = _scatter_call(m2, dst)
    out = _merge2_call(p2, t2, easum, We2,
                       att2.reshape(1, -1), bias2.reshape(1, -1),
                       Wh1, bh1.reshape(1, -1), Wh2.reshape(1, -1),
                       bh2.reshape(1, 1))
    return out[0][:, 0]


# speedup vs baseline: 31.0513x; 1.0017x over previous
import functools

import jax
import jax.numpy as jnp
from jax import lax
from jax.experimental import pallas as pl
from jax.experimental.pallas import tpu as pltpu
from jax.experimental.pallas import tpu_sc as plsc

_N = 10000
_E = 320000
_EPS = 1e-16
_SLOPE = 0.2

_NWORK = 32
_PERW = _E // _NWORK
_K = 80
_NCHUNK = _PERW // _K
_NPAD = 10240
_RPT = _NPAD // 16
_RW = 128

_BN = 400
_BE = 4000


def _lin_kernel(x_ref, wl_ref, bl_ref, wr_ref, br_ref, xl_ref, xr_ref):
    xv = x_ref[...]
    xl_ref[...] = jnp.dot(xv, wl_ref[...], preferred_element_type=jnp.float32) + bl_ref[...]
    xr_ref[...] = jnp.dot(xv, wr_ref[...], preferred_element_type=jnp.float32) + br_ref[...]


def _lin_call(x, wl, bl, wr, br):
    din, dout = wl.shape
    return pl.pallas_call(
        _lin_kernel,
        grid=(_N // _BN,),
        in_specs=[
            pl.BlockSpec((_BN, din), lambda i: (i, 0)),
            pl.BlockSpec((din, dout), lambda i: (0, 0)),
            pl.BlockSpec((1, dout), lambda i: (0, 0)),
            pl.BlockSpec((din, dout), lambda i: (0, 0)),
            pl.BlockSpec((1, dout), lambda i: (0, 0)),
        ],
        out_specs=[
            pl.BlockSpec((_BN, dout), lambda i: (i, 0)),
            pl.BlockSpec((_BN, dout), lambda i: (i, 0)),
        ],
        out_shape=[
            jax.ShapeDtypeStruct((_N, dout), jnp.float32),
            jax.ShapeDtypeStruct((_N, dout), jnp.float32),
        ],
    )(x, wl, bl, wr, br)


_GG = 2


def _gather_call(xl, xr, src, dst):
    d = xl.shape[1]
    mesh = plsc.VectorSubcoreMesh(core_axis_name="c", subcore_axis_name="s")

    scratch = [
        pltpu.VMEM((_PERW,), jnp.int32),
        pltpu.VMEM((_PERW,), jnp.int32),
    ]
    scratch += [pltpu.VMEM((_K, d), jnp.float32) for _ in range(4 * _GG)]
    scratch += [pltpu.SemaphoreType.DMA, pltpu.SemaphoreType.DMA]

    @functools.partial(
        pl.kernel, mesh=mesh,
        out_type=(jax.ShapeDtypeStruct((_E, d), jnp.float32),
                  jax.ShapeDtypeStruct((_E, d), jnp.float32)),
        scratch_types=scratch,
    )
    def gather_kernel(xl_hbm, xr_hbm, src_hbm, dst_hbm, gxl_hbm, gxr_hbm,
                      *bufs):
        srcall, dstall = bufs[0], bufs[1]
        sets = (bufs[2:2 + 2 * _GG], bufs[2 + 2 * _GG:2 + 4 * _GG])
        semg, semw = bufs[-2], bufs[-1]
        cid = lax.axis_index("c")
        sid = lax.axis_index("s")
        base0 = (cid * 16 + sid) * _PERW

        pltpu.sync_copy(src_hbm.at[pl.ds(base0, _PERW)], srcall)
        pltpu.sync_copy(dst_hbm.at[pl.ds(base0, _PERW)], dstall)

        def fire_gathers(c0, bset):
            gd = []
            for s in range(_GG):
                o = (c0 + s) * _K
                gd.append(pltpu.async_copy(
                    xl_hbm.at[srcall.at[pl.ds(o, _K)]], bset[2 * s], semg))
                gd.append(pltpu.async_copy(
                    xr_hbm.at[dstall.at[pl.ds(o, _K)]], bset[2 * s + 1], semg))
            return gd

        def fire_writes(c0, bset):
            wd = []
            for s in range(_GG):
                b = base0 + (c0 + s) * _K
                wd.append(pltpu.async_copy(bset[2 * s], gxl_hbm.at[pl.ds(b, _K)], semw))
                wd.append(pltpu.async_copy(bset[2 * s + 1], gxr_hbm.at[pl.ds(b, _K)], semw))
            return wd

        def drain(ds):
            for c in ds:
                c.wait()

        def fire_one(ci, bset):
            return [pltpu.async_copy(
                        xl_hbm.at[srcall.at[pl.ds(ci * _K, _K)]], bset[0], semg),
                    pltpu.async_copy(
                        xr_hbm.at[dstall.at[pl.ds(ci * _K, _K)]], bset[1], semg)]

        def write_one(ci, bset):
            b = base0 + ci * _K
            return [pltpu.async_copy(bset[0], gxl_hbm.at[pl.ds(b, _K)], semw),
                    pltpu.async_copy(bset[1], gxr_hbm.at[pl.ds(b, _K)], semw)]

        npair = _NCHUNK // (2 * _GG)
        rem = _NCHUNK % (2 * _GG)

        drain(fire_gathers(0, sets[0]))
        fire_writes(_GG, sets[1])

        def body(i, carry):
            c0 = i * 2 * _GG
            c1 = c0 + _GG
            drain(fire_writes(_GG, sets[1]) if False else [])
            for _ in range(2 * _GG):
                pltpu.make_async_copy(sets[1][0], gxl_hbm.at[pl.ds(base0, _K)],
                                      semw).wait()
            w0 = fire_writes(c0, sets[0])
            g1 = fire_gathers(c1, sets[1])
            drain(g1)
            drain(w0)
            fire_writes(c1, sets[1])

            @pl.when(i < npair - 1)
            def _():
                drain(fire_gathers(c0 + 2 * _GG, sets[0]))

            return carry

        lax.fori_loop(0, npair, body, 0)
        for _ in range(2 * _GG):
            pltpu.make_async_copy(sets[1][0], gxl_hbm.at[pl.ds(base0, _K)],
                                  semw).wait()
        for t in range(rem):
            ci = npair * 2 * _GG + t
            drain(fire_one(ci, sets[0]))
            drain(write_one(ci, sets[0]))

    return gather_kernel(xl, xr, src, dst)


def _scatter_call(wm, dst):
    d = 128
    mesh = plsc.VectorSubcoreMesh(core_axis_name="c", subcore_axis_name="s")

    @functools.partial(
        pl.kernel, mesh=mesh,
        out_type=jax.ShapeDtypeStruct((2, _NPAD, d), jnp.float32),
        scratch_types=[
            pltpu.VMEM((_K,), jnp.int32),
            pltpu.VMEM((_K,), jnp.int32),
            pltpu.VMEM((_K, d), jnp.float32),
            pltpu.VMEM((_K, d), jnp.float32),
            pltpu.VMEM((_RW, d), jnp.float32),
            pltpu.VMEM_SHARED((_NPAD, d), jnp.float32),
            pltpu.SemaphoreType.DMA,
            pltpu.SemaphoreType.DMA,
        ],
    )
    def scatter_kernel(wm_hbm, dst_hbm, zw_hbm, wp_hbm,
                       dstv0, dstv1, wv0, wv1, wbv, shw, semr, sema):
        cid = lax.axis_index("c")
        sid = lax.axis_index("s")
        r0 = sid * _RPT

        pltpu.sync_copy(zw_hbm, wbv)

        def zbody(t, carry):
            pltpu.sync_copy(wbv, shw.at[pl.ds(r0 + t * _RW, _RW)])
            return carry

        lax.fori_loop(0, _RPT // _RW, zbody, 0)
        plsc.subcore_barrier()

        base0 = (cid * 16 + sid) * _PERW

        def do_pair(c0, pair):
            descs = []
            for s, (dv, wv) in enumerate(pair):
                b = base0 + (c0 + s) * _K
                descs.append(pltpu.async_copy(dst_hbm.at[pl.ds(b, _K)], dv, semr))
                descs.append(pltpu.async_copy(wm_hbm.at[pl.ds(b, _K)], wv, semr))
            adds = []
            for i, (dv, wv) in enumerate(pair):
                descs[2 * i].wait()
                descs[2 * i + 1].wait()
                adds.append(pltpu.async_copy(wv, shw.at[dv], sema, add=True))
            for c in adds:
                c.wait()

        def chunk(g, carry):
            do_pair(g * 2, [(dstv0, wv0), (dstv1, wv1)])
            return carry

        lax.fori_loop(0, _NCHUNK // 2, chunk, 0)
        if _NCHUNK % 2:
            do_pair(_NCHUNK - 1, [(dstv0, wv0)])
        plsc.subcore_barrier()

        def wb_body(t, carry):
            r = r0 + t * _RW
            pltpu.sync_copy(shw.at[pl.ds(r, _RW)], wbv)
            pltpu.sync_copy(wbv, wp_hbm.at[cid, pl.ds(r, _RW)])
            return carry

        lax.fori_loop(0, _RPT // _RW, wb_body, 0)

    zw = jnp.zeros((_RW, d), jnp.float32)
    wp = scatter_kernel(wm, dst, zw)
    return wp[:, :_N]


def _edge1_kernel(gxl_ref, gxr_ref, ea_ref, we_ref, att_ref,
                  ma_ref, mb_ref, easum_ref):
    i = pl.program_id(0)
    ea = ea_ref[...]
    eaw = jnp.dot(ea, we_ref[...], preferred_element_type=jnp.float32)
    gxl = gxl_ref[...]
    u = gxl + gxr_ref[...] + eaw
    lr = jnp.maximum(u, _SLOPE * u)
    prod = lr * att_ref[...]
    e0 = jnp.exp(jnp.sum(prod[:, :64], axis=1, keepdims=True))
    e1 = jnp.exp(jnp.sum(prod[:, 64:], axis=1, keepdims=True))
    pad = jnp.zeros((e0.shape[0], 63), jnp.float32)
    ma_ref[...] = jnp.concatenate([e0 * gxl[:, :64], e0, pad], axis=1)
    mb_ref[...] = jnp.concatenate([e1 * gxl[:, 64:], e1, pad], axis=1)

    @pl.when(i == 0)
    def _():
        easum_ref[...] = jnp.zeros_like(easum_ref)

    easum_ref[...] += jnp.sum(ea, axis=0, keepdims=True)


def _edge1_call(gxl, gxr, ea, we, att):
    return pl.pallas_call(
        _edge1_kernel,
        grid=(_E // _BE,),
        in_specs=[
            pl.BlockSpec((_BE, 128), lambda i: (i, 0)),
            pl.BlockSpec((_BE, 128), lambda i: (i, 0)),
            pl.BlockSpec((_BE, 4), lambda i: (i, 0)),
            pl.BlockSpec((4, 128), lambda i: (0, 0)),
            pl.BlockSpec((1, 128), lambda i: (0, 0)),
        ],
        out_specs=[
            pl.BlockSpec((_BE, 128), lambda i: (i, 0)),
            pl.BlockSpec((_BE, 128), lambda i: (i, 0)),
            pl.BlockSpec((1, 4), lambda i: (0, 0)),
        ],
        out_shape=[
            jax.ShapeDtypeStruct((_E, 128), jnp.float32),
            jax.ShapeDtypeStruct((_E, 128), jnp.float32),
            jax.ShapeDtypeStruct((1, 4), jnp.float32),
        ],
    )(gxl, gxr, ea, we, att)


def _edge2_kernel(gsrc_ref, gdst_ref, ea_ref, we_ref, att_ref, m_ref):
    eaw = jnp.dot(ea_ref[...], we_ref[...], preferred_element_type=jnp.float32)
    xls = gsrc_ref[:, :64]
    u = xls + gdst_ref[:, 64:] + eaw
    lr = jnp.maximum(u, _SLOPE * u)
    e0 = jnp.exp(jnp.sum(lr * att_ref[...], axis=1, keepdims=True))
    pad = jnp.zeros((e0.shape[0], 63), jnp.float32)
    m_ref[...] = jnp.concatenate([e0 * xls, e0, pad], axis=1)


def _edge2_call(gsrc, gdst, ea, we, att):
    return pl.pallas_call(
        _edge2_kernel,
        grid=(_E // _BE,),
        in_specs=[
            pl.BlockSpec((_BE, 128), lambda i: (i, 0)),
            pl.BlockSpec((_BE, 128), lambda i: (i, 0)),
            pl.BlockSpec((_BE, 4), lambda i: (i, 0)),
            pl.BlockSpec((4, 64), lambda i: (0, 0)),
            pl.BlockSpec((1, 64), lambda i: (0, 0)),
        ],
        out_specs=[pl.BlockSpec((_BE, 128), lambda i: (i, 0))],
        out_shape=[jax.ShapeDtypeStruct((_E, 128), jnp.float32)],
    )(gsrc, gdst, ea, we, att)


def _merge1_kernel(pa_ref, pb_ref, xl_ref, xr_ref, easum_ref,
                   we_ref, att_ref, bias_ref, wl2_ref, bl2_ref, wr2_ref,
                   br2_ref, t2_ref):
    eawm = jnp.dot(easum_ref[...] / _E, we_ref[...],
                   preferred_element_type=jnp.float32)
    xl = xl_ref[...]
    u = xl + xr_ref[...] + eawm
    lr = jnp.maximum(u, _SLOPE * u)
    prod = lr * att_ref[...]
    e0 = jnp.exp(jnp.sum(prod[:, :64], axis=1, keepdims=True))
    e1 = jnp.exp(jnp.sum(prod[:, 64:], axis=1, keepdims=True))
    pa = pa_ref[0] + pa_ref[1]
    pb = pb_ref[0] + pb_ref[1]
    w0 = pa[:, :64] + e0 * xl[:, :64]
    w1 = pb[:, :64] + e1 * xl[:, 64:]
    s0 = pa[:, 64:65] + e0
    s1 = pb[:, 64:65] + e1
    h = jnp.concatenate([w0 / (s0 + _EPS), w1 / (s1 + _EPS)], axis=1)
    h = jnp.maximum(h + bias_ref[...], 0.0)
    xl2 = jnp.dot(h, wl2_ref[...], preferred_element_type=jnp.float32) + bl2_ref[...]
    xr2 = jnp.dot(h, wr2_ref[...], preferred_element_type=jnp.float32) + br2_ref[...]
    t2_ref[...] = jnp.concatenate([xl2, xr2], axis=1)


def _merge1_call(pa, pb, xl, xr, easum, we, att, bias, wl2, bl2,
                 wr2, br2):
    return pl.pallas_call(
        _merge1_kernel,
        grid=(_N // _BN,),
        in_specs=[
            pl.BlockSpec((2, _BN, 128), lambda i: (0, i, 0)),
            pl.BlockSpec((2, _BN, 128), lambda i: (0, i, 0)),
            pl.BlockSpec((_BN, 128), lambda i: (i, 0)),
            pl.BlockSpec((_BN, 128), lambda i: (i, 0)),
            pl.BlockSpec((1, 4), lambda i: (0, 0)),
            pl.BlockSpec((4, 128), lambda i: (0, 0)),
            pl.BlockSpec((1, 128), lambda i: (0, 0)),
            pl.BlockSpec((1, 128), lambda i: (0, 0)),
            pl.BlockSpec((128, 64), lambda i: (0, 0)),
            pl.BlockSpec((1, 64), lambda i: (0, 0)),
            pl.BlockSpec((128, 64), lambda i: (0, 0)),
            pl.BlockSpec((1, 64), lambda i: (0, 0)),
        ],
        out_specs=[pl.BlockSpec((_BN, 128), lambda i: (i, 0))],
        out_shape=[jax.ShapeDtypeStruct((_N, 128), jnp.float32)],
    )(pa, pb, xl, xr, easum, we, att, bias, wl2, bl2, wr2, br2)


def _merge2_kernel(p_ref, t2_ref, easum_ref, we_ref, att_ref,
                   bias_ref, wh1_ref, bh1_ref, wh2_ref, bh2_ref, out_ref):
    eawm = jnp.dot(easum_ref[...] / _E, we_ref[...],
                   preferred_element_type=jnp.float32)
    xl = t2_ref[:, :64]
    u = xl + t2_ref[:, 64:] + eawm
    lr = jnp.maximum(u, _SLOPE * u)
    e0 = jnp.exp(jnp.sum(lr * att_ref[...], axis=1, keepdims=True))
    p = p_ref[0] + p_ref[1]
    w = p[:, :64] + e0 * xl
    s0 = p[:, 64:65] + e0
    h = jnp.maximum(w / (s0 + _EPS) + bias_ref[...], 0.0)
    h = jnp.maximum(jnp.dot(h, wh1_ref[...], preferred_element_type=jnp.float32)
                    + bh1_ref[...], 0.0)
    out_ref[...] = jnp.sum(h * wh2_ref[...], axis=1, keepdims=True) + bh2_ref[...]


def _merge2_call(p, t2, easum, we, att, bias, wh1, bh1, wh2t, bh2):
    return pl.pallas_call(
        _merge2_kernel,
        grid=(_N // _BN,),
        in_specs=[
            pl.BlockSpec((2, _BN, 128), lambda i: (0, i, 0)),
            pl.BlockSpec((_BN, 128), lambda i: (i, 0)),
            pl.BlockSpec((1, 4), lambda i: (0, 0)),
            pl.BlockSpec((4, 64), lambda i: (0, 0)),
            pl.BlockSpec((1, 64), lambda i: (0, 0)),
            pl.BlockSpec((1, 64), lambda i: (0, 0)),
            pl.BlockSpec((64, 64), lambda i: (0, 0)),
            pl.BlockSpec((1, 64), lambda i: (0, 0)),
            pl.BlockSpec((1, 64), lambda i: (0, 0)),
            pl.BlockSpec((1, 1), lambda i: (0, 0)),
        ],
        out_specs=[pl.BlockSpec((_BN, 1), lambda i: (i, 0))],
        out_shape=[jax.ShapeDtypeStruct((_N, 1), jnp.float32)],
    )(p, t2, easum, we, att, bias, wh1, bh1, wh2t, bh2)


def kernel(x, edge_index, edge_attr, Wl1, bl1, Wr1, br1, We1, att1, bias1,
           Wl2, bl2, Wr2, br2, We2, att2, bias2, Wh1, bh1, Wh2, bh2):
    src = edge_index[0]
    dst = edge_index[1]

    xl1, xr1 = _lin_call(x, Wl1, bl1.reshape(1, -1), Wr1, br1.reshape(1, -1))
    gxl, gxr = _gather_call(xl1, xr1, src, dst)
    ma, mb, easum = _edge1_call(gxl, gxr, edge_attr, We1,
                                att1.reshape(1, -1))
    pa = _scatter_call(ma, dst)
    pb = _scatter_call(mb, dst)
    t2 = _merge1_call(pa, pb, xl1, xr1, easum, We1,
                      att1.reshape(1, -1), bias1.reshape(1, -1),
                      Wl2, bl2.reshape(1, -1), Wr2, br2.reshape(1, -1))[0]

    gsrc, gdst = _gather_call(t2, t2, src, dst)
    m2 = _edge2_call(gsrc, gdst, edge_attr, We2, att2.reshape(1, -1))[0]
    p2 = _scatter_call(m2, dst)
    out = _merge2_call(p2, t2, easum, We2,
                       att2.reshape(1, -1), bias2.reshape(1, -1),
                       Wh1, bh1.reshape(1, -1), Wh2.reshape(1, -1),
                       bh2.reshape(1, 1))
    return out[0][:, 0]
